# Initial kernel scaffold; baseline (speedup 1.0000x reference)
import jax
import jax.numpy as jnp
from jax.experimental import pallas as pl


def kernel(x, edge_index, W1, b1, W2, b2):
    raise NotImplementedError("write your pallas kernel here")


# speedup vs baseline: 9.8782x; 9.8782x over previous
import jax
import jax.numpy as jnp
from jax import lax
from jax.experimental import pallas as pl
from jax.experimental.pallas import tpu as pltpu
from jax.experimental.pallas import tpu_sc as plsc

NC = 2
NS = 16
NW = NC * NS
DEGW = 16


def _sc_agg_colsplit(n, d2, nb, k):
  rows_t = n // NS
  mesh = plsc.VectorSubcoreMesh(core_axis_name="c", subcore_axis_name="s")

  out_type = (jax.ShapeDtypeStruct((NC, n, d2), jnp.float32),
              jax.ShapeDtypeStruct((NC, n, DEGW), jnp.float32))
  scratch = [
      pltpu.VMEM((nb, k), jnp.int32),
      pltpu.VMEM((nb, k), jnp.int32),
      pltpu.VMEM((2, k, d2), jnp.float32),
      pltpu.VMEM((k, DEGW), jnp.float32),
      pltpu.VMEM_SHARED((n, d2), jnp.float32),
      pltpu.VMEM_SHARED((n, DEGW), jnp.float32),
      pltpu.SemaphoreType.DMA,
      pltpu.SemaphoreType.DMA,
  ]

  def body(table, srcr, dstr, zrow, zdeg, ones_h, psum, pdeg,
           src_v, dst_v, gbuf, ones_v, acc, dacc, sem0, sem1):
    c = lax.axis_index("c")
    s = lax.axis_index("s")
    base = s * rows_t
    sems = (sem0, sem1)

    pltpu.sync_copy(zrow, acc.at[pl.ds(base, rows_t)])
    pltpu.sync_copy(zdeg, dacc.at[pl.ds(base, rows_t)])
    pltpu.sync_copy(ones_h, ones_v)
    pltpu.sync_copy(srcr.at[c, s], src_v)
    pltpu.sync_copy(dstr.at[s], dst_v)
    plsc.subcore_barrier()

    pltpu.async_copy(table.at[src_v.at[0]], gbuf.at[0], sem0)

    @pl.loop(0, nb, step=2)
    def _(b0):
      for j in range(2):
        b = b0 + j
        pltpu.make_async_copy(table.at[src_v.at[b]], gbuf.at[j],
                              sems[j]).wait()
        nxt = b + 1

        @pl.when(nxt < nb)
        def _():
          pltpu.async_copy(table.at[src_v.at[nxt]], gbuf.at[1 - j],
                           sems[1 - j])

        pltpu.sync_copy(gbuf.at[j], acc.at[dst_v.at[b]], add=True)

        @pl.when(c == j)
        def _():
          pltpu.sync_copy(ones_v, dacc.at[dst_v.at[b]], add=True)

    plsc.subcore_barrier()
    pltpu.sync_copy(acc.at[pl.ds(base, rows_t)],
                    psum.at[c, pl.ds(base, rows_t)])
    pltpu.sync_copy(dacc.at[pl.ds(base, rows_t)],
                    pdeg.at[c, pl.ds(base, rows_t)])

  return pl.kernel(body, out_type=out_type, mesh=mesh,
                   scratch_types=scratch,
                   compiler_params=pltpu.CompilerParams(
                       use_tc_tiling_on_sc=False))


def _sc_agg_edgesplit(n, d, nb, k):
  rows_t = n // NS
  mesh = plsc.VectorSubcoreMesh(core_axis_name="c", subcore_axis_name="s")

  out_type = jax.ShapeDtypeStruct((NC, n, d), jnp.float32)
  scratch = [
      pltpu.VMEM((nb, k), jnp.int32),
      pltpu.VMEM((nb, k), jnp.int32),
      pltpu.VMEM((2, k, d), jnp.float32),
      pltpu.VMEM_SHARED((n, d), jnp.float32),
      pltpu.SemaphoreType.DMA,
      pltpu.SemaphoreType.DMA,
  ]

  def body(table, srcr, dstr, zrow, psum,
           src_v, dst_v, gbuf, acc, sem0, sem1):
    c = lax.axis_index("c")
    s = lax.axis_index("s")
    wid = s * NC + c
    base = s * rows_t
    sems = (sem0, sem1)

    pltpu.sync_copy(zrow, acc.at[pl.ds(base, rows_t)])
    pltpu.sync_copy(srcr.at[wid], src_v)
    pltpu.sync_copy(dstr.at[wid], dst_v)
    plsc.subcore_barrier()

    pltpu.async_copy(table.at[src_v.at[0]], gbuf.at[0], sem0)

    @pl.loop(0, nb, step=2)
    def _(b0):
      for j in range(2):
        b = b0 + j
        pltpu.make_async_copy(table.at[src_v.at[b]], gbuf.at[j],
                              sems[j]).wait()
        nxt = b + 1

        @pl.when(nxt < nb)
        def _():
          pltpu.async_copy(table.at[src_v.at[nxt]], gbuf.at[1 - j],
                           sems[1 - j])

        pltpu.sync_copy(gbuf.at[j], acc.at[dst_v.at[b]], add=True)

    plsc.subcore_barrier()
    pltpu.sync_copy(acc.at[pl.ds(base, rows_t)],
                    psum.at[c, pl.ds(base, rows_t)])

  return pl.kernel(body, out_type=out_type, mesh=mesh,
                   scratch_types=scratch,
                   compiler_params=pltpu.CompilerParams(
                       use_tc_tiling_on_sc=False))


def _tc_layer0(x, psum, pdeg, w1a, w1b, b1, w2a, w2b):
  n, f = x.shape
  o = w2a.shape[1]

  def body(x_ref, ps_ref, pd_ref, w1a_ref, w1b_ref, b1_ref, w2a_ref,
           w2b_ref, q_ref, p_ref):
    deg = pd_ref[0, :n, 0:1] + pd_ref[1, :n, 0:1]
    recip = 1.0 / jnp.maximum(deg, 1.0)
    s = jnp.concatenate([ps_ref[0, :n, :], ps_ref[1, :n, :]], axis=1)
    hn = s * recip
    h1 = jnp.dot(x_ref[...], w1a_ref[...],
                 preferred_element_type=jnp.float32)
    h1 += jnp.dot(hn, w1b_ref[...], preferred_element_type=jnp.float32)
    h1 = jnp.maximum(h1 + b1_ref[...], 0.0)
    q_ref[...] = jnp.dot(h1, w2a_ref[...],
                         preferred_element_type=jnp.float32)
    p_ref[...] = jnp.dot(h1, w2b_ref[...],
                         preferred_element_type=jnp.float32)

  return pl.pallas_call(
      body,
      out_shape=(jax.ShapeDtypeStruct((n, o), jnp.float32),
                 jax.ShapeDtypeStruct((n, o), jnp.float32)),
  )(x, psum, pdeg, w1a, w1b, b1, w2a, w2b)


def _tc_layer1(q, s2, pdeg, b2):
  n, o = q.shape

  def body(q_ref, s2_ref, pd_ref, b2_ref, o_ref):
    deg = pd_ref[0, :n, 0:1] + pd_ref[1, :n, 0:1]
    recip = 1.0 / jnp.maximum(deg, 1.0)
    o_ref[...] = (q_ref[...]
                  + (s2_ref[0, :n, :] + s2_ref[1, :n, :]) * recip
                  + b2_ref[...])

  return pl.pallas_call(
      body, out_shape=jax.ShapeDtypeStruct((n, o), jnp.float32),
  )(q, s2, pdeg, b2)


def kernel(x, edge_index, W1, b1, W2, b2):
  n, f = x.shape
  e = edge_index.shape[1]
  h = W1.shape[0]
  o = W2.shape[0]
  f2 = f // 2
  k = 125
  assert e % (NS * k) == 0 and e % (NW * k) == 0
  nb0 = e // (NS * k)
  nb1 = e // (NW * k)
  assert nb0 % 2 == 0 and nb1 % 2 == 0

  n_pad = -(-n // (NS * 8)) * (NS * 8)
  rows_t = n_pad // NS

  src = edge_index[0].astype(jnp.int32)
  dst = edge_index[1].astype(jnp.int32)
  src0 = jnp.stack([src, src + n]).reshape(NC, NS, nb0, k)
  dst0 = dst.reshape(NS, nb0, k)
  src1 = src.reshape(NW, nb1, k)
  dst1 = dst.reshape(NW, nb1, k)
  x2 = x.reshape(n, NC, f2).transpose(1, 0, 2).reshape(NC * n, f2)

  zrow_f2 = jnp.zeros((rows_t, f2), jnp.float32)
  zrow_o = jnp.zeros((rows_t, o), jnp.float32)
  zdeg = jnp.zeros((rows_t, DEGW), jnp.float32)
  ones_h = jnp.ones((k, DEGW), jnp.float32)

  agg0 = _sc_agg_colsplit(n_pad, f2, nb0, k)
  psum, pdeg = agg0(x2, src0, dst0, zrow_f2, zdeg, ones_h)

  w1a = W1[:, :f].T
  w1b = W1[:, f:].T
  w2a = W2[:, :h].T
  w2b = W2[:, h:].T
  q, p = _tc_layer0(x, psum, pdeg, w1a, w1b, b1.reshape(1, h),
                    w2a, w2b)

  agg1 = _sc_agg_edgesplit(n_pad, o, nb1, k)
  s2 = agg1(p, src1, dst1, zrow_o)

  return _tc_layer1(q, s2, pdeg, b2.reshape(1, o))


# speedup vs baseline: 13.3065x; 1.3470x over previous
import jax
import jax.numpy as jnp
from jax import lax
from jax.experimental import pallas as pl
from jax.experimental.pallas import tpu as pltpu
from jax.experimental.pallas import tpu_sc as plsc

NC = 2
NS = 16
NW = NC * NS
DEGW = 16


def _sc_agg_colsplit(n, d2, nb, k):
  rows_t = n // NS
  mesh = plsc.VectorSubcoreMesh(core_axis_name="c", subcore_axis_name="s")

  out_type = (jax.ShapeDtypeStruct((NC, n, d2), jnp.float32),
              jax.ShapeDtypeStruct((NC, n, DEGW), jnp.float32))
  assert nb % 4 == 0
  scratch = [
      pltpu.VMEM((nb, k), jnp.int32),
      pltpu.VMEM((nb, k), jnp.int32),
      pltpu.VMEM((4, k, d2), jnp.float32),
      pltpu.VMEM((k, DEGW), jnp.float32),
      pltpu.VMEM_SHARED((n, d2), jnp.float32),
      pltpu.VMEM_SHARED((n, DEGW), jnp.float32),
  ] + [pltpu.SemaphoreType.DMA] * 10

  def body(table, srcr, dstr, zrow, zdeg, ones_h, psum, pdeg,
           src_v, dst_v, gbuf, ones_v, acc, dacc, *sems):
    c = lax.axis_index("c")
    s = lax.axis_index("s")
    base = s * rows_t
    sem_g = sems[0:4]
    sem_s = sems[4:8]
    sem_d = sems[8:10]

    pltpu.sync_copy(zrow, acc.at[pl.ds(base, rows_t)])
    pltpu.sync_copy(zdeg, dacc.at[pl.ds(base, rows_t)])
    pltpu.sync_copy(ones_h, ones_v)
    pltpu.sync_copy(srcr.at[c, s], src_v)
    pltpu.sync_copy(dstr.at[s], dst_v)
    plsc.subcore_barrier()

    for j in range(3):
      pltpu.async_copy(table.at[src_v.at[j]], gbuf.at[j], sem_g[j])

    @pl.loop(0, nb, step=4)
    def _(b0):
      for j in range(4):
        b = b0 + j
        pltpu.make_async_copy(table.at[src_v.at[b]], gbuf.at[j],
                              sem_g[j]).wait()
        pltpu.async_copy(gbuf.at[j], acc.at[dst_v.at[b]], sem_s[j],
                         add=True)

        @pl.when(c == j % 2)
        def _():
          jd = j // 2

          @pl.when(b0 > 0)
          def _():
            pltpu.make_async_copy(ones_v, dacc.at[dst_v.at[0]],
                                  sem_d[jd]).wait()

          pltpu.async_copy(ones_v, dacc.at[dst_v.at[b]], sem_d[jd],
                           add=True)

        jn = (j + 3) % 4
        nxt = b + 3

        @pl.when(b > 0)
        def _():
          pltpu.make_async_copy(gbuf.at[jn], acc.at[dst_v.at[0]],
                                sem_s[jn]).wait()

        @pl.when(nxt < nb)
        def _():
          pltpu.async_copy(table.at[src_v.at[nxt]], gbuf.at[jn],
                           sem_g[jn])

    pltpu.make_async_copy(gbuf.at[3], acc.at[dst_v.at[0]], sem_s[3]).wait()
    for jd in range(2):
      pltpu.make_async_copy(ones_v, dacc.at[dst_v.at[0]],
                            sem_d[jd]).wait()
    plsc.subcore_barrier()
    pltpu.sync_copy(acc.at[pl.ds(base, rows_t)],
                    psum.at[c, pl.ds(base, rows_t)])
    pltpu.sync_copy(dacc.at[pl.ds(base, rows_t)],
                    pdeg.at[c, pl.ds(base, rows_t)])

  return pl.kernel(body, out_type=out_type, mesh=mesh,
                   scratch_types=scratch,
                   compiler_params=pltpu.CompilerParams(
                       use_tc_tiling_on_sc=False))


def _sc_agg_edgesplit(n, d, nb, k):
  rows_t = n // NS
  mesh = plsc.VectorSubcoreMesh(core_axis_name="c", subcore_axis_name="s")

  assert nb % 4 == 0
  out_type = jax.ShapeDtypeStruct((NC, n, d), jnp.float32)
  scratch = [
      pltpu.VMEM((nb, k), jnp.int32),
      pltpu.VMEM((nb, k), jnp.int32),
      pltpu.VMEM((4, k, d), jnp.float32),
      pltpu.VMEM_SHARED((n, d), jnp.float32),
  ] + [pltpu.SemaphoreType.DMA] * 8

  def body(table, srcr, dstr, zrow, psum,
           src_v, dst_v, gbuf, acc, *sems):
    c = lax.axis_index("c")
    s = lax.axis_index("s")
    wid = s * NC + c
    base = s * rows_t
    sem_g = sems[0:4]
    sem_s = sems[4:8]

    pltpu.sync_copy(zrow, acc.at[pl.ds(base, rows_t)])
    pltpu.sync_copy(srcr.at[wid], src_v)
    pltpu.sync_copy(dstr.at[wid], dst_v)
    plsc.subcore_barrier()

    for j in range(3):
      pltpu.async_copy(table.at[src_v.at[j]], gbuf.at[j], sem_g[j])

    @pl.loop(0, nb, step=4)
    def _(b0):
      for j in range(4):
        b = b0 + j
        pltpu.make_async_copy(table.at[src_v.at[b]], gbuf.at[j],
                              sem_g[j]).wait()
        pltpu.async_copy(gbuf.at[j], acc.at[dst_v.at[b]], sem_s[j],
                         add=True)
        jn = (j + 3) % 4
        nxt = b + 3

        @pl.when(b > 0)
        def _():
          pltpu.make_async_copy(gbuf.at[jn], acc.at[dst_v.at[0]],
                                sem_s[jn]).wait()

        @pl.when(nxt < nb)
        def _():
          pltpu.async_copy(table.at[src_v.at[nxt]], gbuf.at[jn],
                           sem_g[jn])

    pltpu.make_async_copy(gbuf.at[3], acc.at[dst_v.at[0]], sem_s[3]).wait()
    plsc.subcore_barrier()
    pltpu.sync_copy(acc.at[pl.ds(base, rows_t)],
                    psum.at[c, pl.ds(base, rows_t)])

  return pl.kernel(body, out_type=out_type, mesh=mesh,
                   scratch_types=scratch,
                   compiler_params=pltpu.CompilerParams(
                       use_tc_tiling_on_sc=False))


def _tc_layer0(x, psum, pdeg, w1a, w1b, b1, w2a, w2b):
  n, f = x.shape
  o = w2a.shape[1]

  def body(x_ref, ps_ref, pd_ref, w1a_ref, w1b_ref, b1_ref, w2a_ref,
           w2b_ref, q_ref, p_ref):
    deg = pd_ref[0, :n, 0:1] + pd_ref[1, :n, 0:1]
    recip = 1.0 / jnp.maximum(deg, 1.0)
    s = jnp.concatenate([ps_ref[0, :n, :], ps_ref[1, :n, :]], axis=1)
    hn = s * recip
    h1 = jnp.dot(x_ref[...], w1a_ref[...],
                 preferred_element_type=jnp.float32)
    h1 += jnp.dot(hn, w1b_ref[...], preferred_element_type=jnp.float32)
    h1 = jnp.maximum(h1 + b1_ref[...], 0.0)
    q_ref[...] = jnp.dot(h1, w2a_ref[...],
                         preferred_element_type=jnp.float32)
    p_ref[...] = jnp.dot(h1, w2b_ref[...],
                         preferred_element_type=jnp.float32)

  return pl.pallas_call(
      body,
      out_shape=(jax.ShapeDtypeStruct((n, o), jnp.float32),
                 jax.ShapeDtypeStruct((n, o), jnp.float32)),
  )(x, psum, pdeg, w1a, w1b, b1, w2a, w2b)


def _tc_layer1(q, s2, pdeg, b2):
  n, o = q.shape

  def body(q_ref, s2_ref, pd_ref, b2_ref, o_ref):
    deg = pd_ref[0, :n, 0:1] + pd_ref[1, :n, 0:1]
    recip = 1.0 / jnp.maximum(deg, 1.0)
    o_ref[...] = (q_ref[...]
                  + (s2_ref[0, :n, :] + s2_ref[1, :n, :]) * recip
                  + b2_ref[...])

  return pl.pallas_call(
      body, out_shape=jax.ShapeDtypeStruct((n, o), jnp.float32),
  )(q, s2, pdeg, b2)


def kernel(x, edge_index, W1, b1, W2, b2):
  n, f = x.shape
  e = edge_index.shape[1]
  h = W1.shape[0]
  o = W2.shape[0]
  f2 = f // 2
  k = 125
  assert e % (NS * k) == 0 and e % (NW * k) == 0
  nb0 = e // (NS * k)
  nb1 = e // (NW * k)
  assert nb0 % 2 == 0 and nb1 % 2 == 0

  n_pad = -(-n // (NS * 8)) * (NS * 8)
  rows_t = n_pad // NS

  src = edge_index[0].astype(jnp.int32)
  dst = edge_index[1].astype(jnp.int32)
  src0 = jnp.stack([src, src + n]).reshape(NC, NS, nb0, k)
  dst0 = dst.reshape(NS, nb0, k)
  src1 = src.reshape(NW, nb1, k)
  dst1 = dst.reshape(NW, nb1, k)
  x2 = x.reshape(n, NC, f2).transpose(1, 0, 2).reshape(NC * n, f2)

  zrow_f2 = jnp.zeros((rows_t, f2), jnp.float32)
  zrow_o = jnp.zeros((rows_t, o), jnp.float32)
  zdeg = jnp.zeros((rows_t, DEGW), jnp.float32)
  ones_h = jnp.ones((k, DEGW), jnp.float32)

  agg0 = _sc_agg_colsplit(n_pad, f2, nb0, k)
  psum, pdeg = agg0(x2, src0, dst0, zrow_f2, zdeg, ones_h)

  w1a = W1[:, :f].T
  w1b = W1[:, f:].T
  w2a = W2[:, :h].T
  w2b = W2[:, h:].T
  q, p = _tc_layer0(x, psum, pdeg, w1a, w1b, b1.reshape(1, h),
                    w2a, w2b)

  agg1 = _sc_agg_edgesplit(n_pad, o, nb1, k)
  s2 = agg1(p, src1, dst1, zrow_o)

  return _tc_layer1(q, s2, pdeg, b2.reshape(1, o))
